# baseline (speedup 1.0000x reference)
import jax
import jax.numpy as jnp
from jax.experimental import pallas as pl


def _s_kernel(x_ref, w1_ref, w2_ref, o_ref):
    w = jnp.dot(w1_ref[...], w2_ref[...], preferred_element_type=jnp.float32)
    o_ref[...] = jnp.dot(x_ref[...], w, preferred_element_type=jnp.float32)


def _mm_kernel(a_ref, b_ref, o_ref):
    o_ref[...] = jnp.dot(a_ref[...], b_ref[...], preferred_element_type=jnp.float32)


def _adj_matmul(adj, b, bm):
    n, k = adj.shape
    _, d = b.shape
    return pl.pallas_call(
        _mm_kernel,
        grid=(n // bm,),
        in_specs=[
            pl.BlockSpec((bm, k), lambda i: (i, 0)),
            pl.BlockSpec((k, d), lambda i: (0, 0)),
        ],
        out_specs=pl.BlockSpec((bm, d), lambda i: (i, 0)),
        out_shape=jax.ShapeDtypeStruct((n, d), jnp.float32),
    )(adj, b)


def kernel(adj, x, W_enc, W_mean):
    n, d_in = x.shape
    d_emb = W_mean.shape[1]
    bm_s = 2000
    s = pl.pallas_call(
        _s_kernel,
        grid=(n // bm_s,),
        in_specs=[
            pl.BlockSpec((bm_s, d_in), lambda i: (i, 0)),
            pl.BlockSpec((d_in, d_emb), lambda i: (0, 0)),
            pl.BlockSpec((d_emb, d_emb), lambda i: (0, 0)),
        ],
        out_specs=pl.BlockSpec((bm_s, d_emb), lambda i: (i, 0)),
        out_shape=jax.ShapeDtypeStruct((n, d_emb), jnp.float32),
    )(x, W_enc, W_mean)
    t = _adj_matmul(adj, s, 80)
    return _adj_matmul(adj, t, 80)


# speedup vs baseline: 1.3930x; 1.3930x over previous
import jax
import jax.numpy as jnp
from jax.experimental import pallas as pl
from jax.experimental import pallas as _pl
from jax.experimental.pallas import tpu as pltpu

_BM = 200
_BS = 2000


def _fused_kernel(x_ref, w1_ref, w2_ref, adj_ref, o_ref, s_ref, t_ref,
                  *, nb, ns):
    i = pl.program_id(0)

    @pl.when(i < ns)
    def _s_phase():
        w = jnp.dot(w1_ref[...], w2_ref[...],
                    preferred_element_type=jnp.float32)
        s_ref[pl.ds(i * _BS, _BS), :] = jnp.dot(
            x_ref[...], w, preferred_element_type=jnp.float32)

    @pl.when((i >= ns) & (i < ns + nb))
    def _pass1():
        t_ref[pl.ds((i - ns) * _BM, _BM), :] = jnp.dot(
            adj_ref[...], s_ref[...], preferred_element_type=jnp.float32)

    @pl.when(i >= ns + nb)
    def _pass2():
        o_ref[...] = jnp.dot(adj_ref[...], t_ref[...],
                             preferred_element_type=jnp.float32)


def kernel(adj, x, W_enc, W_mean):
    n, d_in = x.shape
    d_emb = W_mean.shape[1]
    nb = n // _BM
    ns = n // _BS
    import functools
    body = functools.partial(_fused_kernel, nb=nb, ns=ns)

    def adj_map(i):
        p1 = jnp.clip(i - ns, 0, nb - 1)
        p2 = i - (ns + nb)
        return (jnp.where(i < ns + nb, p1, p2), 0)

    return pl.pallas_call(
        body,
        grid=(ns + 2 * nb,),
        in_specs=[
            pl.BlockSpec((_BS, d_in), lambda i: (jnp.minimum(i, ns - 1), 0)),
            pl.BlockSpec((d_in, d_emb), lambda i: (0, 0)),
            pl.BlockSpec((d_emb, d_emb), lambda i: (0, 0)),
            pl.BlockSpec((_BM, n), adj_map),
        ],
        out_specs=pl.BlockSpec(
            (_BM, d_emb), lambda i: (jnp.maximum(i - (ns + nb), 0), 0)),
        out_shape=jax.ShapeDtypeStruct((n, d_emb), jnp.float32),
        scratch_shapes=[
            pltpu.VMEM((n, d_emb), jnp.float32),
            pltpu.VMEM((n, d_emb), jnp.float32),
        ],
    )(x, W_enc, W_mean, adj)
